# baseline (speedup 1.0000x reference)
import functools

import jax
import jax.numpy as jnp
from jax import lax
from jax.experimental import pallas as pl
from jax.experimental.pallas import tpu as pltpu
from jax.experimental.pallas import tpu_sc as plsc

B, N, F = 16, 2048, 2048
L = 16
NV = N // L
PV = 7
K = 100
N_TILE = 1024

SC_B = 4
SC_ROWS = SC_B * N
TC_ROWS = (B - SC_B) * N
NW = 32
RPT = SC_ROWS // NW
G = 16
NG = RPT // G

_NEG = -3e38
_POS = 3e38


def _gemv_body(xa_ref, xb_ref, wa_ref, wb_ref, b_ref, s_ref):
    dn = (((1,), (1,)), ((), ()))
    res = lax.dot_general(wa_ref[...], xa_ref[...], dn,
                          preferred_element_type=jnp.float32)
    res = res + lax.dot_general(wb_ref[...], xb_ref[...], dn,
                                preferred_element_type=jnp.float32)
    s_ref[...] = res + b_ref[0]


def _gemv(xf, W1, b1):
    half = F // 2
    return pl.pallas_call(
        _gemv_body,
        grid=(TC_ROWS // N_TILE,),
        in_specs=[
            pl.BlockSpec((N_TILE, half), lambda n: (n, 0)),
            pl.BlockSpec((N_TILE, half), lambda n: (n, 1)),
            pl.BlockSpec((1, half), lambda n: (0, 0)),
            pl.BlockSpec((1, half), lambda n: (0, 1)),
            pl.BlockSpec(memory_space=pltpu.SMEM),
        ],
        out_specs=pl.BlockSpec((1, N_TILE), lambda n: (0, n)),
        out_shape=jax.ShapeDtypeStruct((1, TC_ROWS), jnp.float32),
    )(xf, xf, W1, W1, b1)


@functools.partial(
    pl.kernel,
    out_type=jax.ShapeDtypeStruct((SC_ROWS,), jnp.float32),
    mesh=plsc.VectorSubcoreMesh(core_axis_name="c", subcore_axis_name="s"),
    compiler_params=pltpu.CompilerParams(needs_layout_passes=False),
    scratch_types=[
        pltpu.VMEM((G, F), jnp.float32),
        pltpu.VMEM((G, F), jnp.float32),
        pltpu.VMEM((F,), jnp.float32),
        pltpu.VMEM((L,), jnp.float32),
        pltpu.VMEM((RPT,), jnp.float32),
        pltpu.SemaphoreType.DMA,
        pltpu.SemaphoreType.DMA,
    ],
)
def _sc_gemv(x_hbm, w_hbm, b1_hbm, out_hbm,
             grp0_v, grp1_v, w_v, b1_v, res_v, sem0, sem1):
    cid = lax.axis_index("c")
    sid = lax.axis_index("s")
    wid = sid * 2 + cid
    base = TC_ROWS + wid * RPT
    pltpu.sync_copy(w_hbm, w_v)
    pltpu.sync_copy(b1_hbm, b1_v)
    grps = (grp0_v, grp1_v)
    sems = (sem0, sem1)
    lane = lax.broadcasted_iota(jnp.int32, (L,), 0)

    def start(g, b):
        pltpu.make_async_copy(
            x_hbm.at[pl.ds(base + g * G, G)], grps[b], sems[b]).start()

    def wait(b):
        pltpu.make_async_copy(
            x_hbm.at[pl.ds(base, G)], grps[b], sems[b]).wait()

    def compute_group(grp_v, gg):
        rvec = jnp.zeros((L,), jnp.float32)
        for j0 in range(0, G, 4):
            zero = jnp.zeros((L,), jnp.float32)
            init = (zero,) * 16

            def fchunk(c, accs, j0=j0):
                new = list(accs)
                for a in range(4):
                    off = (c * 4 + a) * L
                    wv = w_v[pl.ds(off, L)]
                    for r in range(4):
                        idx = r * 4 + a
                        new[idx] = new[idx] + grp_v[j0 + r, pl.ds(off, L)] * wv
                return tuple(new)

            accs = lax.fori_loop(0, F // L // 4, fchunk, init)
            for r in range(4):
                a0, a1, a2, a3 = accs[r * 4:(r + 1) * 4]
                t = (a0 + a1) + (a2 + a3)
                sj = jnp.sum(t)
                rvec = jnp.where(lane == (j0 + r), jnp.full((L,), sj, jnp.float32), rvec)
        res_v[pl.ds(gg * L, L)] = rvec + b1_v[...]

    start(0, 0)
    start(1, 1)

    def pair(i, _):
        g = i * 2
        for b in range(2):
            wait(b)
            compute_group(grps[b], g + b)

            @pl.when(g + b + 2 < NG)
            def _(g=g, b=b):
                start(g + b + 2, b)
        return 0

    lax.fori_loop(0, NG // 2, pair, 0)
    pltpu.sync_copy(res_v, out_hbm.at[pl.ds(wid * RPT, RPT)])


def _merge_desc(a, b):
    rb = lax.rev(b, (0,))
    hi = jnp.maximum(a, rb)
    lo = jnp.minimum(a, rb)
    hi, _ = plsc.sort_key_val(hi, hi, descending=True)
    lo, _ = plsc.sort_key_val(lo, lo, descending=True)
    return hi, lo


def _merge_asc(a, b):
    rb = lax.rev(b, (0,))
    lo = jnp.minimum(a, rb)
    hi = jnp.maximum(a, rb)
    lo, _ = plsc.sort_key_val(lo, lo)
    hi, _ = plsc.sort_key_val(hi, hi)
    return lo, hi


@functools.partial(
    pl.kernel,
    out_type=(
        jax.ShapeDtypeStruct((B, L), jnp.float32),
        jax.ShapeDtypeStruct((B, L), jnp.float32),
    ),
    mesh=plsc.VectorSubcoreMesh(core_axis_name="c", subcore_axis_name="s"),
    compiler_params=pltpu.CompilerParams(needs_layout_passes=False),
    scratch_types=[
        pltpu.VMEM((N,), jnp.float32),
        pltpu.VMEM((2 * PV * L,), jnp.float32),
        pltpu.VMEM((L,), jnp.float32),
        pltpu.VMEM((L,), jnp.float32),
    ],
)
def _sc_topk(s_hbm, w2_hbm, bias_hbm, prob_hbm, hat_hbm,
             row_v, w2_v, bias_v, out_v):
    cid = lax.axis_index("c")
    sid = lax.axis_index("s")

    @pl.when(cid == 0)
    def _():
        pltpu.sync_copy(s_hbm.at[sid], row_v)
        pltpu.sync_copy(w2_hbm, w2_v)
        pltpu.sync_copy(bias_hbm, bias_v)

        init = ((jnp.full((L,), _NEG, jnp.float32),) * PV
                + (jnp.full((L,), _POS, jnp.float32),) * PV)

        def body(i, pools):
            v = row_v[pl.ds(i * L, L)]
            vd, _ = plsc.sort_key_val(v, v, descending=True)
            new = []
            carry = vd
            for k in range(PV):
                hi, carry = _merge_desc(pools[k], carry)
                new.append(hi)
            va, _ = plsc.sort_key_val(v, v)
            carry = va
            for k in range(PV):
                lo, carry = _merge_asc(pools[PV + k], carry)
                new.append(lo)
            return tuple(new)

        pools = lax.fori_loop(0, NV, body, init)

        acc = jnp.zeros((L,), jnp.float32)
        for k in range(2 * PV):
            acc = acc + pools[k] * w2_v[pl.ds(k * L, L)]
        total = jnp.sum(acc)

        logit = jnp.full((L,), total) + bias_v[...]
        prob = 1.0 / (1.0 + jnp.exp(-logit))
        out_v[...] = prob
        pltpu.sync_copy(out_v, prob_hbm.at[sid])
        out_v[...] = jnp.where(prob >= 0.5, 1.0, 0.0).astype(jnp.float32)
        pltpu.sync_copy(out_v, hat_hbm.at[sid])


def kernel(x, W1, b1, W2, b2):
    xf = x.reshape(B * N, F)
    b1v = jnp.broadcast_to(b1.astype(jnp.float32), (L,))
    s_tc = _gemv(xf, W1, b1)
    s_sc = _sc_gemv(xf, W1.reshape(F), b1v)
    s = jnp.concatenate(
        [s_tc.reshape(B - SC_B, N), s_sc.reshape(SC_B, N)], axis=0)
    zeros12 = jnp.zeros((PV * L - K,), jnp.float32)
    w2pad = jnp.concatenate([W2[0, :K], zeros12, W2[0, K:], zeros12])
    biasv = jnp.broadcast_to(b2.astype(jnp.float32), (L,))
    prob, hat = _sc_topk(s, w2pad, biasv)
    return (prob[:, :1], hat[:, :1])


# speedup vs baseline: 1.0499x; 1.0499x over previous
import functools

import jax
import jax.numpy as jnp
from jax import lax
from jax.experimental import pallas as pl
from jax.experimental.pallas import tpu as pltpu
from jax.experimental.pallas import tpu_sc as plsc

B, N, F = 16, 2048, 2048
L = 16
NV = N // L
PV = 7
K = 100
N_TILE = 1024

SC_B = 5
SC_ROWS = SC_B * N
TC_ROWS = (B - SC_B) * N
NW = 32
RPT = SC_ROWS // NW
G = 16
NG = RPT // G

_NEG = -3e38
_POS = 3e38


def _gemv_body(xa_ref, xb_ref, wa_ref, wb_ref, b_ref, s_ref):
    dn = (((1,), (1,)), ((), ()))
    res = lax.dot_general(wa_ref[...], xa_ref[...], dn,
                          preferred_element_type=jnp.float32)
    res = res + lax.dot_general(wb_ref[...], xb_ref[...], dn,
                                preferred_element_type=jnp.float32)
    s_ref[...] = res + b_ref[0]


def _gemv(xf, W1, b1):
    half = F // 2
    return pl.pallas_call(
        _gemv_body,
        grid=(TC_ROWS // N_TILE,),
        in_specs=[
            pl.BlockSpec((N_TILE, half), lambda n: (n, 0)),
            pl.BlockSpec((N_TILE, half), lambda n: (n, 1)),
            pl.BlockSpec((1, half), lambda n: (0, 0)),
            pl.BlockSpec((1, half), lambda n: (0, 1)),
            pl.BlockSpec(memory_space=pltpu.SMEM),
        ],
        out_specs=pl.BlockSpec((1, N_TILE), lambda n: (0, n)),
        out_shape=jax.ShapeDtypeStruct((1, TC_ROWS), jnp.float32),
    )(xf, xf, W1, W1, b1)


@functools.partial(
    pl.kernel,
    out_type=jax.ShapeDtypeStruct((SC_ROWS,), jnp.float32),
    mesh=plsc.VectorSubcoreMesh(core_axis_name="c", subcore_axis_name="s"),
    compiler_params=pltpu.CompilerParams(needs_layout_passes=False),
    scratch_types=[
        pltpu.VMEM((G, F), jnp.float32),
        pltpu.VMEM((G, F), jnp.float32),
        pltpu.VMEM((F,), jnp.float32),
        pltpu.VMEM((L,), jnp.float32),
        pltpu.VMEM((RPT,), jnp.float32),
        pltpu.SemaphoreType.DMA,
        pltpu.SemaphoreType.DMA,
    ],
)
def _sc_gemv(x_hbm, w_hbm, b1_hbm, out_hbm,
             grp0_v, grp1_v, w_v, b1_v, res_v, sem0, sem1):
    cid = lax.axis_index("c")
    sid = lax.axis_index("s")
    wid = sid * 2 + cid
    base = TC_ROWS + wid * RPT
    pltpu.sync_copy(w_hbm, w_v)
    pltpu.sync_copy(b1_hbm, b1_v)
    grps = (grp0_v, grp1_v)
    sems = (sem0, sem1)
    lane = lax.broadcasted_iota(jnp.int32, (L,), 0)

    def start(g, b):
        pltpu.make_async_copy(
            x_hbm.at[pl.ds(base + g * G, G)], grps[b], sems[b]).start()

    def wait(b):
        pltpu.make_async_copy(
            x_hbm.at[pl.ds(base, G)], grps[b], sems[b]).wait()

    def compute_group(grp_v, gg):
        rvec = jnp.zeros((L,), jnp.float32)
        for j0 in range(0, G, 4):
            zero = jnp.zeros((L,), jnp.float32)
            init = (zero,) * 16

            def fchunk(c, accs, j0=j0):
                new = list(accs)
                for a in range(4):
                    off = (c * 4 + a) * L
                    wv = w_v[pl.ds(off, L)]
                    for r in range(4):
                        idx = r * 4 + a
                        new[idx] = new[idx] + grp_v[j0 + r, pl.ds(off, L)] * wv
                return tuple(new)

            accs = lax.fori_loop(0, F // L // 4, fchunk, init)
            for r in range(4):
                a0, a1, a2, a3 = accs[r * 4:(r + 1) * 4]
                t = (a0 + a1) + (a2 + a3)
                sj = jnp.sum(t)
                rvec = jnp.where(lane == (j0 + r), jnp.full((L,), sj, jnp.float32), rvec)
        res_v[pl.ds(gg * L, L)] = rvec + b1_v[...]

    start(0, 0)
    start(1, 1)

    def pair(i, _):
        g = i * 2
        for b in range(2):
            wait(b)
            compute_group(grps[b], g + b)

            @pl.when(g + b + 2 < NG)
            def _(g=g, b=b):
                start(g + b + 2, b)
        return 0

    lax.fori_loop(0, NG // 2, pair, 0)
    pltpu.sync_copy(res_v, out_hbm.at[pl.ds(wid * RPT, RPT)])


def _merge_desc(a, b):
    rb = lax.rev(b, (0,))
    hi = jnp.maximum(a, rb)
    lo = jnp.minimum(a, rb)
    hi, _ = plsc.sort_key_val(hi, hi, descending=True)
    lo, _ = plsc.sort_key_val(lo, lo, descending=True)
    return hi, lo


def _merge_asc(a, b):
    rb = lax.rev(b, (0,))
    lo = jnp.minimum(a, rb)
    hi = jnp.maximum(a, rb)
    lo, _ = plsc.sort_key_val(lo, lo)
    hi, _ = plsc.sort_key_val(hi, hi)
    return lo, hi


@functools.partial(
    pl.kernel,
    out_type=(
        jax.ShapeDtypeStruct((B,), jnp.float32),
        jax.ShapeDtypeStruct((B,), jnp.float32),
    ),
    mesh=plsc.VectorSubcoreMesh(core_axis_name="c", subcore_axis_name="s"),
    compiler_params=pltpu.CompilerParams(needs_layout_passes=False),
    scratch_types=[
        pltpu.VMEM((N,), jnp.float32),
        pltpu.VMEM((2 * PV * L,), jnp.float32),
        pltpu.VMEM((L,), jnp.float32),
        pltpu.VMEM((L,), jnp.float32),
        pltpu.VMEM((B, L), jnp.float32),
        pltpu.VMEM_SHARED((B, L), jnp.float32),
    ],
)
def _sc_topk(s_tc_hbm, s_sc_hbm, w2_hbm, bias_hbm, prob_hbm, hat_hbm,
             row_v, w2_v, bias_v, st_v, all_v, shared):
    cid = lax.axis_index("c")
    sid = lax.axis_index("s")

    @pl.when(cid == 0)
    def _():
        @pl.when(sid < B - SC_B)
        def _():
            pltpu.sync_copy(s_tc_hbm.at[0, pl.ds(sid * N, N)], row_v)

        @pl.when(sid >= B - SC_B)
        def _():
            pltpu.sync_copy(
                s_sc_hbm.at[pl.ds((sid - (B - SC_B)) * N, N)], row_v)

        pltpu.sync_copy(w2_hbm, w2_v)

        init = ((jnp.full((L,), _NEG, jnp.float32),) * PV
                + (jnp.full((L,), _POS, jnp.float32),) * PV)

        def body(i, pools):
            v = row_v[pl.ds(i * L, L)]
            vd, _ = plsc.sort_key_val(v, v, descending=True)
            new = []
            carry = vd
            for k in range(PV):
                hi, carry = _merge_desc(pools[k], carry)
                new.append(hi)
            va, _ = plsc.sort_key_val(v, v)
            carry = va
            for k in range(PV):
                lo, carry = _merge_asc(pools[PV + k], carry)
                new.append(lo)
            return tuple(new)

        pools = lax.fori_loop(0, NV, body, init)

        acc = jnp.zeros((L,), jnp.float32)
        for k in range(2 * PV):
            acc = acc + pools[k] * w2_v[pl.ds(k * L, L)]
        total = jnp.sum(acc)

        st_v[...] = jnp.full((L,), total, jnp.float32)
        pltpu.sync_copy(st_v, shared.at[sid])
        plsc.subcore_barrier()

        @pl.when(sid == 0)
        def _():
            pltpu.sync_copy(bias_hbm, bias_v)
            pltpu.sync_copy(shared, all_v)
            ii = lax.broadcasted_iota(jnp.int32, (L,), 0)
            diag = plsc.load_gather(all_v, [ii, ii])
            logit = diag + bias_v[...]
            prob = 1.0 / (1.0 + jnp.exp(-logit))
            st_v[...] = prob
            pltpu.sync_copy(st_v, prob_hbm)
            st_v[...] = jnp.where(prob >= 0.5, 1.0, 0.0).astype(jnp.float32)
            pltpu.sync_copy(st_v, hat_hbm)


def kernel(x, W1, b1, W2, b2):
    xf = x.reshape(B * N, F)
    b1v = jnp.broadcast_to(b1.astype(jnp.float32), (L,))
    s_tc = _gemv(xf, W1, b1)
    s_sc = _sc_gemv(xf, W1.reshape(F), b1v)
    zeros12 = jnp.zeros((PV * L - K,), jnp.float32)
    w2pad = jnp.concatenate([W2[0, :K], zeros12, W2[0, K:], zeros12])
    biasv = jnp.broadcast_to(b2.astype(jnp.float32), (L,))
    prob, hat = _sc_topk(s_tc, s_sc, w2pad, biasv)
    return (prob.reshape(B, 1), hat.reshape(B, 1))
